# Initial kernel scaffold; baseline (speedup 1.0000x reference)
import functools
import jax
import jax.numpy as jnp
from jax import lax
from jax.experimental import pallas as pl
from jax.experimental.pallas import tpu as pltpu
from jax.experimental.pallas import tpu_sc as plsc

N = 10000
E = 160000
NF = 32

NC = 2
NS = 16
NW = NC * NS
CB = 128
NCH = 40
EPW = CB * NCH
EP = EPW * NW
NACC = 10240

_mesh = functools.partial(
    plsc.VectorSubcoreMesh,
    core_axis_name="c", subcore_axis_name="s", num_cores=NC, num_subcores=NS)


def _wid():
    return lax.axis_index("s") * NC + lax.axis_index("c")


def _make_gather(dims, sels):
    n = len(dims)
    scratch = [pltpu.VMEM((NCH, CB), jnp.int32), pltpu.VMEM((NCH, CB), jnp.int32)]
    scratch += [pltpu.VMEM((CB, d), jnp.float32) for d in dims]
    scratch += [pltpu.SemaphoreType.DMA for _ in dims]

    def body(*refs):
        tabs = refs[:n]
        idxr_h, idxc_h = refs[n], refs[n + 1]
        outs = refs[n + 2:2 * n + 2]
        idxr_v, idxc_v = refs[2 * n + 2], refs[2 * n + 3]
        bufs = refs[2 * n + 4:3 * n + 4]
        sems = refs[3 * n + 4:4 * n + 4]
        w = _wid()
        pltpu.sync_copy(idxr_h.at[w], idxr_v)
        pltpu.sync_copy(idxc_h.at[w], idxc_v)

        def step(j, carry):
            for i in range(n):
                iv = idxr_v if sels[i] == 0 else idxc_v
                pltpu.async_copy(tabs[i].at[iv.at[j]], bufs[i], sems[i])
            for i in range(n):
                iv = idxr_v if sels[i] == 0 else idxc_v
                pltpu.make_async_copy(tabs[i].at[iv.at[j]], bufs[i],
                                      sems[i]).wait()
                pltpu.sync_copy(bufs[i], outs[i].at[pl.ds(w * EPW + j * CB, CB)])
            return carry

        lax.fori_loop(0, NCH, step, 0)

    out_type = tuple(jax.ShapeDtypeStruct((EP, d), jnp.float32) for d in dims)
    return pl.kernel(body, out_type=out_type, mesh=_mesh(),
                     scratch_types=tuple(scratch))


def _make_scatter(with_counts):
    scratch = [
        pltpu.VMEM_SHARED((NACC, NF), jnp.float32),
        pltpu.VMEM((NCH, CB), jnp.int32),
        pltpu.VMEM((CB, NF), jnp.float32),
    ]
    if with_counts:
        scratch += [pltpu.VMEM_SHARED((NACC, 16), jnp.float32),
                    pltpu.VMEM((CB, 16), jnp.float32)]

    def body(*refs):
        if with_counts:
            (vals_h, idx_h, z32_h, z16_h, ones_h, out_h, outc_h,
             acc_sh, idx_v, val_v, accc_sh, ones_v) = refs
        else:
            vals_h, idx_h, z32_h, out_h, acc_sh, idx_v, val_v = refs
        c = lax.axis_index("c")
        s = lax.axis_index("s")
        w = _wid()

        @pl.when(s == 0)
        def _init():
            pltpu.sync_copy(z32_h, acc_sh)
            if with_counts:
                pltpu.sync_copy(z16_h, accc_sh)

        pltpu.sync_copy(idx_h.at[w], idx_v)
        if with_counts:
            pltpu.sync_copy(ones_h, ones_v)
        plsc.subcore_barrier()

        def step(j, carry):
            pltpu.sync_copy(vals_h.at[pl.ds(w * EPW + j * CB, CB)], val_v)
            pltpu.sync_copy(val_v, acc_sh.at[idx_v.at[j]], add=True)
            if with_counts:
                pltpu.sync_copy(ones_v, accc_sh.at[idx_v.at[j]], add=True)
            return carry

        lax.fori_loop(0, NCH, step, 0)
        plsc.subcore_barrier()

        @pl.when(s == 0)
        def _flush():
            pltpu.sync_copy(acc_sh, out_h.at[c])
            if with_counts:
                pltpu.sync_copy(accc_sh, outc_h.at[c])

    out_type = [jax.ShapeDtypeStruct((NC, NACC, NF), jnp.float32)]
    if with_counts:
        out_type.append(jax.ShapeDtypeStruct((NC, NACC, 16), jnp.float32))
    return pl.kernel(body, out_type=tuple(out_type), mesh=_mesh(),
                     scratch_types=tuple(scratch))


def _col(a, i):
    return a[:, i:i + 1]


def _qmul(q, r):
    q0, q1, q2, q3 = _col(q, 0), _col(q, 1), _col(q, 2), _col(q, 3)
    r0, r1, r2, r3 = _col(r, 0), _col(r, 1), _col(r, 2), _col(r, 3)
    w = r0 * q0 - r1 * q1 - r2 * q2 - r3 * q3
    x = r0 * q1 + r1 * q0 - r2 * q3 + r3 * q2
    y = r0 * q2 + r1 * q3 + r2 * q0 - r3 * q1
    z = r0 * q3 - r1 * q2 + r2 * q1 + r3 * q0
    return jnp.concatenate([w, x, y, z], axis=1)


def _qinv(q):
    return jnp.concatenate([_col(q, 0), -_col(q, 1), -_col(q, 2), -_col(q, 3)],
                           axis=1)


def _edge_blockspecs(widths, be):
    return [pl.BlockSpec((be, w), lambda i: (i, 0)) for w in widths]


def _full_spec(shape):
    nd = len(shape)
    return pl.BlockSpec(shape, lambda i: (0,) * nd)


BE = 4096
BN = 2048


def _tc_call(body, in_arrays, in_specs, out_shapes, out_specs, grid):
    return pl.pallas_call(
        body,
        grid=(grid,),
        in_specs=in_specs,
        out_specs=out_specs,
        out_shape=out_shapes,
    )(*in_arrays)


def _edge1_body(a0, b0, ea, g1, h1, w1ae, b1a, w1b, b1b, w2aeam, w2ae1, b2a,
                e1_o, d2_o, ginv_o):
    xi = a0[:, 0:4]
    gqr = a0[:, 4:8]
    xj = b0[:, 0:4]
    gqc = b0[:, 4:8]
    W = _qmul(ea[...], xi)
    eam = _qmul(_qinv(xj), W)
    ginv_o[...] = _qinv(_qmul(gqc, _qinv(gqr)))
    t = jax.nn.relu(g1[...] + h1[...] + jnp.dot(eam, w1ae[...],
                    preferred_element_type=jnp.float32) + b1a[...])
    e1 = jnp.dot(t, w1b[...], preferred_element_type=jnp.float32) + b1b[...]
    e1_o[...] = e1
    d2_o[...] = (jnp.dot(eam, w2aeam[...], preferred_element_type=jnp.float32)
                 + jnp.dot(jax.nn.relu(e1), w2ae1[...],
                           preferred_element_type=jnp.float32) + b2a[...])


def _edge_mid_body(g, h, d, ep, wkb, bkb, wna_ek, wna_ep, bna, ek_o, dn_o):
    t = jax.nn.relu(g[...] + h[...] + d[...])
    ek = jnp.dot(t, wkb[...], preferred_element_type=jnp.float32) + bkb[...]
    ek_o[...] = ek
    dn_o[...] = (jnp.dot(jax.nn.relu(ek), wna_ek[...],
                         preferred_element_type=jnp.float32)
                 + jnp.dot(jax.nn.relu(ep[...]), wna_ep[...],
                           preferred_element_type=jnp.float32) + bna[...])


def _edge4_body(g, h, d, w4b, b4b, e4_o):
    t = jax.nn.relu(g[...] + h[...] + d[...])
    e4_o[...] = jnp.dot(t, w4b[...], preferred_element_type=jnp.float32) + b4b[...]


def _node1_body(p0, p1, c0, c1, wg, wh, x1_o, g_o, h_o, invc_o):
    cnt = c0[...] + c1[...]
    invc = 1.0 / jnp.maximum(cnt, 1.0)
    invc_o[...] = invc
    x1 = jax.nn.relu((p0[...] + p1[...]) * invc[:, 0:1])
    x1_o[...] = x1
    g_o[...] = jnp.dot(x1, wg[...], preferred_element_type=jnp.float32)
    h_o[...] = jnp.dot(x1, wh[...], preferred_element_type=jnp.float32)


def _node_mid_body(p0, p1, invc, xp, wg_a, wg_b, wh_a, wh_b, xk_o, g_o, h_o):
    xk = jax.nn.relu((p0[...] + p1[...]) * invc[:, 0:1])
    xk_o[...] = xk
    g_o[...] = (jnp.dot(xk, wg_a[...], preferred_element_type=jnp.float32)
                + jnp.dot(xp[...], wg_b[...], preferred_element_type=jnp.float32))
    h_o[...] = (jnp.dot(xk, wh_a[...], preferred_element_type=jnp.float32)
                + jnp.dot(xp[...], wh_b[...], preferred_element_type=jnp.float32))


def _node4_body(p0, p1, invc, xorg, wl, bl, t5_o):
    x4 = jax.nn.relu((p0[...] + p1[...]) * invc[:, 0:1])
    xq = (jnp.dot(x4, wl[...], preferred_element_type=jnp.float32) + bl[...]
          + xorg[...])
    nrm = jnp.sqrt(jnp.sum(xq * xq, axis=1, keepdims=True))
    xn = xq / jnp.maximum(nrm, 1e-12)
    z = jnp.zeros_like(xn[:, 0:3])
    t5_o[...] = jnp.concatenate([xn, invc[:, 0:1], z], axis=1)


def _loss_body(a5, b5, ginv, beta, out):
    i = pl.program_id(0)

    @pl.when(i == 0)
    def _z():
        out[...] = jnp.zeros_like(out)

    x_row = a5[:, 0:4]
    invc_r = a5[:, 4:5]
    x_col = b5[:, 0:4]
    em = _qmul(x_col, _qinv(x_row))
    l1 = _qmul(ginv[...], em)
    nrm = jnp.sqrt(jnp.sum(l1 * l1, axis=1, keepdims=True))
    l1 = l1 / jnp.maximum(nrm, 1e-12)
    alpha = 0.05
    nn0 = jnp.minimum(1.0 - l1[:, 0:1], 1.0 + l1[:, 0:1])
    nnv = (jnp.abs(nn0[:, 0]) + jnp.abs(l1[:, 1]) + jnp.abs(l1[:, 2])
           + jnp.abs(l1[:, 3])) * beta[:, 0]
    le = jnp.where(nnv < alpha, 0.5 * nnv * nnv / alpha, nnv - 0.5 * alpha)
    out[0, 0] += jnp.sum(le * invc_r[:, 0]) * (1.0 / N)


_gather4 = _make_gather((8, 8, NF, NF), (0, 1, 0, 1))
_gather2 = _make_gather((NF, NF), (0, 1))
_gather2s = _make_gather((8, 8), (0, 1))
_scatter_c = _make_scatter(True)
_scatter = _make_scatter(False)


def kernel(x_org, edge_index, edge_attr, gt_q, beta,
           W1a, b1a, W1b, b1b, W2a, b2a, W2b, b2b,
           W3a, b3a, W3b, b3b, W4a, b4a, W4b, b4b, Wl, bl):
    f32 = jnp.float32
    row = edge_index[0].astype(jnp.int32)
    col = edge_index[1].astype(jnp.int32)
    padE = EP - E
    rowg = jnp.pad(row, (0, padE)).reshape(NW, NCH, CB)
    colg = jnp.pad(col, (0, padE)).reshape(NW, NCH, CB)
    rows = jnp.pad(row, (0, padE), constant_values=N).reshape(NW, NCH, CB)

    z32 = jnp.zeros((NACC, NF), f32)
    z16 = jnp.zeros((NACC, 16), f32)
    ones16 = jnp.ones((CB, 16), f32)
    eap = jnp.pad(edge_attr[:, :4], ((0, padE), (0, 0)))
    betap = jnp.pad(beta, (0, padE)).reshape(EP, 1)

    T0 = jnp.pad(jnp.concatenate([x_org, gt_q], axis=1),
                 ((0, NACC - N), (0, 0)))
    xorgp = jnp.pad(x_org, ((0, NACC - N), (0, 0)))

    def _prep_body(xo, wg, wh, g_o, h_o):
        g_o[...] = jnp.dot(xo[...], wg[...], preferred_element_type=jnp.float32)
        h_o[...] = jnp.dot(xo[...], wh[...], preferred_element_type=jnp.float32)

    g1t, h1t = _tc_call(
        _prep_body,
        [xorgp, W1a[0:4], W1a[4:8]],
        [pl.BlockSpec((BN, 4), lambda i: (i, 0)), _full_spec((4, NF)),
         _full_spec((4, NF))],
        (jax.ShapeDtypeStruct((NACC, NF), f32),
         jax.ShapeDtypeStruct((NACC, NF), f32)),
        [pl.BlockSpec((BN, NF), lambda i: (i, 0))] * 2,
        NACC // BN)

    A0, B0, G1, H1 = _gather4(T0, T0, g1t, h1t, rowg, colg)

    nb_e = EP // BE
    e1, D2, ginv = _tc_call(
        _edge1_body,
        [A0, B0, eap, G1, H1, W1a[8:12], b1a.reshape(1, NF), W1b,
         b1b.reshape(1, NF), W2a[64:68], W2a[68:100], b2a.reshape(1, NF)],
        _edge_blockspecs([8, 8, 4, NF, NF], BE)
        + [_full_spec((4, NF)), _full_spec((1, NF)), _full_spec((NF, NF)),
           _full_spec((1, NF)), _full_spec((4, NF)), _full_spec((NF, NF)),
           _full_spec((1, NF))],
        (jax.ShapeDtypeStruct((EP, NF), f32), jax.ShapeDtypeStruct((EP, NF), f32),
         jax.ShapeDtypeStruct((EP, 4), f32)),
        _edge_blockspecs([NF, NF, 4], BE),
        nb_e)

    P, C = _scatter_c(e1, rows, z32, z16, ones16)
    x1t, g2t, h2t, invc = _tc_call(
        _node1_body,
        [P[0], P[1], C[0], C[1], W2a[0:32], W2a[32:64]],
        [pl.BlockSpec((BN, NF), lambda i: (i, 0))] * 2
        + [pl.BlockSpec((BN, 16), lambda i: (i, 0))] * 2
        + [_full_spec((NF, NF))] * 2,
        (jax.ShapeDtypeStruct((NACC, NF), f32),
         jax.ShapeDtypeStruct((NACC, NF), f32),
         jax.ShapeDtypeStruct((NACC, NF), f32),
         jax.ShapeDtypeStruct((NACC, 16), f32)),
        [pl.BlockSpec((BN, NF), lambda i: (i, 0))] * 3
        + [pl.BlockSpec((BN, 16), lambda i: (i, 0))],
        NACC // BN)

    eprev = e1
    xprev = x1t
    gt, ht = g2t, h2t
    Ws = {2: (W2b, b2b, W3a[128:160], W3a[160:192], b3a,
              W3a[0:32], W3a[32:64], W3a[64:96], W3a[96:128]),
          3: (W3b, b3b, W4a[128:160], W4a[160:192], b4a,
              W4a[0:32], W4a[32:64], W4a[64:96], W4a[96:128])}
    for k in (2, 3):
        Gk, Hk = _gather2(gt, ht, rowg, colg)
        wkb, bkb, wna_ek, wna_ep, bna, wga, wgb, wha, whb = Ws[k]
        ek, Dn = _tc_call(
            _edge_mid_body,
            [Gk, Hk, D2, eprev, wkb, bkb.reshape(1, NF), wna_ek, wna_ep,
             bna.reshape(1, NF)],
            _edge_blockspecs([NF, NF, NF, NF], BE)
            + [_full_spec((NF, NF)), _full_spec((1, NF)), _full_spec((NF, NF)),
               _full_spec((NF, NF)), _full_spec((1, NF))],
            (jax.ShapeDtypeStruct((EP, NF), f32),
             jax.ShapeDtypeStruct((EP, NF), f32)),
            _edge_blockspecs([NF, NF], BE),
            nb_e)
        Pk = _scatter(ek, rows, z32)[0]
        xk, gnt, hnt = _tc_call(
            _node_mid_body,
            [Pk[0], Pk[1], invc, xprev, wga, wgb, wha, whb],
            [pl.BlockSpec((BN, NF), lambda i: (i, 0))] * 2
            + [pl.BlockSpec((BN, 16), lambda i: (i, 0))]
            + [pl.BlockSpec((BN, NF), lambda i: (i, 0))]
            + [_full_spec((NF, NF))] * 4,
            (jax.ShapeDtypeStruct((NACC, NF), f32),
             jax.ShapeDtypeStruct((NACC, NF), f32),
             jax.ShapeDtypeStruct((NACC, NF), f32)),
            [pl.BlockSpec((BN, NF), lambda i: (i, 0))] * 3,
            NACC // BN)
        eprev = ek
        D2 = Dn
        xprev = xk
        gt, ht = gnt, hnt

    G4, H4 = _gather2(gt, ht, rowg, colg)
    e4 = _tc_call(
        _edge4_body,
        [G4, H4, D2, W4b, b4b.reshape(1, NF)],
        _edge_blockspecs([NF, NF, NF], BE)
        + [_full_spec((NF, NF)), _full_spec((1, NF))],
        jax.ShapeDtypeStruct((EP, NF), f32),
        _edge_blockspecs([NF], BE)[0],
        nb_e)
    P4 = _scatter(e4, rows, z32)[0]
    T5 = _tc_call(
        _node4_body,
        [P4[0], P4[1], invc, xorgp, Wl, bl.reshape(1, 4)],
        [pl.BlockSpec((BN, NF), lambda i: (i, 0))] * 2
        + [pl.BlockSpec((BN, 16), lambda i: (i, 0)),
           pl.BlockSpec((BN, 4), lambda i: (i, 0)),
           _full_spec((NF, 4)), _full_spec((1, 4))],
        jax.ShapeDtypeStruct((NACC, 8), f32),
        pl.BlockSpec((BN, 8), lambda i: (i, 0)),
        NACC // BN)

    A5, B5 = _gather2s(T5, T5, rowg, colg)
    lsum = _tc_call(
        _loss_body,
        [A5, B5, ginv, betap],
        _edge_blockspecs([8, 8, 4, 1], BE),
        jax.ShapeDtypeStruct((1, 1), f32),
        pl.BlockSpec((1, 1), lambda i: (0, 0)),
        nb_e)

    x = T5[:N, 0:4]
    return (x, lsum[0, 0], beta)


# speedup vs baseline: 1.3671x; 1.3671x over previous
import functools
import jax
import jax.numpy as jnp
from jax import lax
from jax.experimental import pallas as pl
from jax.experimental.pallas import tpu as pltpu
from jax.experimental.pallas import tpu_sc as plsc

N = 10000
E = 160000
NF = 32

NC = 2
NS = 16
NW = NC * NS
CB = 128
NCH = 40
EPW = CB * NCH
EP = EPW * NW
NACC = 10240

_mesh = functools.partial(
    plsc.VectorSubcoreMesh,
    core_axis_name="c", subcore_axis_name="s", num_cores=NC, num_subcores=NS)


def _wid():
    return lax.axis_index("s") * NC + lax.axis_index("c")


def _make_gather(dims, sels):
    n = len(dims)
    scratch = [pltpu.VMEM((NCH, CB), jnp.int32), pltpu.VMEM((NCH, CB), jnp.int32)]
    scratch += [pltpu.VMEM((CB, d), jnp.float32) for d in dims]
    scratch += [pltpu.SemaphoreType.DMA for _ in dims]

    def body(*refs):
        tabs = refs[:n]
        idxr_h, idxc_h = refs[n], refs[n + 1]
        outs = refs[n + 2:2 * n + 2]
        idxr_v, idxc_v = refs[2 * n + 2], refs[2 * n + 3]
        bufs = refs[2 * n + 4:3 * n + 4]
        sems = refs[3 * n + 4:4 * n + 4]
        w = _wid()
        pltpu.sync_copy(idxr_h.at[w], idxr_v)
        pltpu.sync_copy(idxc_h.at[w], idxc_v)

        def step(j, carry):
            handles = []
            for i in range(n):
                iv = idxr_v if sels[i] == 0 else idxc_v
                handles.append(
                    pltpu.async_copy(tabs[i].at[iv.at[j]], bufs[i], sems[i]))
            for i in range(n):
                handles[i].wait()
                pltpu.sync_copy(bufs[i], outs[i].at[pl.ds(w * EPW + j * CB, CB)])
            return carry

        lax.fori_loop(0, NCH, step, 0)

    out_type = tuple(jax.ShapeDtypeStruct((EP, d), jnp.float32) for d in dims)
    return pl.kernel(body, out_type=out_type, mesh=_mesh(),
                     scratch_types=tuple(scratch),
                     compiler_params=pltpu.CompilerParams(
                         use_tc_tiling_on_sc=False))


def _make_scatter(with_counts):
    scratch = [
        pltpu.VMEM_SHARED((NACC, NF), jnp.float32),
        pltpu.VMEM((NCH, CB), jnp.int32),
        pltpu.VMEM((CB, NF), jnp.float32),
    ]
    if with_counts:
        scratch += [pltpu.VMEM_SHARED((NACC, 16), jnp.float32),
                    pltpu.VMEM((CB, 16), jnp.float32)]

    def body(*refs):
        if with_counts:
            (vals_h, idx_h, z32_h, z16_h, ones_h, out_h, outc_h,
             acc_sh, idx_v, val_v, accc_sh, ones_v) = refs
        else:
            vals_h, idx_h, z32_h, out_h, acc_sh, idx_v, val_v = refs
        c = lax.axis_index("c")
        s = lax.axis_index("s")
        w = _wid()

        @pl.when(s == 0)
        def _init():
            pltpu.sync_copy(z32_h, acc_sh)
            if with_counts:
                pltpu.sync_copy(z16_h, accc_sh)

        pltpu.sync_copy(idx_h.at[w], idx_v)
        if with_counts:
            pltpu.sync_copy(ones_h, ones_v)
        plsc.subcore_barrier()

        def step(j, carry):
            pltpu.sync_copy(vals_h.at[pl.ds(w * EPW + j * CB, CB)], val_v)
            pltpu.sync_copy(val_v, acc_sh.at[idx_v.at[j]], add=True)
            if with_counts:
                pltpu.sync_copy(ones_v, accc_sh.at[idx_v.at[j]], add=True)
            return carry

        lax.fori_loop(0, NCH, step, 0)
        plsc.subcore_barrier()

        @pl.when(s == 0)
        def _flush():
            pltpu.sync_copy(acc_sh, out_h.at[c])
            if with_counts:
                pltpu.sync_copy(accc_sh, outc_h.at[c])

    out_type = [jax.ShapeDtypeStruct((NC, NACC, NF), jnp.float32)]
    if with_counts:
        out_type.append(jax.ShapeDtypeStruct((NC, NACC, 16), jnp.float32))
    return pl.kernel(body, out_type=tuple(out_type), mesh=_mesh(),
                     scratch_types=tuple(scratch),
                     compiler_params=pltpu.CompilerParams(
                         use_tc_tiling_on_sc=False))


def _col(a, i):
    return a[:, i:i + 1]


def _qmul(q, r):
    q0, q1, q2, q3 = _col(q, 0), _col(q, 1), _col(q, 2), _col(q, 3)
    r0, r1, r2, r3 = _col(r, 0), _col(r, 1), _col(r, 2), _col(r, 3)
    w = r0 * q0 - r1 * q1 - r2 * q2 - r3 * q3
    x = r0 * q1 + r1 * q0 - r2 * q3 + r3 * q2
    y = r0 * q2 + r1 * q3 + r2 * q0 - r3 * q1
    z = r0 * q3 - r1 * q2 + r2 * q1 + r3 * q0
    return jnp.concatenate([w, x, y, z], axis=1)


def _qinv(q):
    return jnp.concatenate([_col(q, 0), -_col(q, 1), -_col(q, 2), -_col(q, 3)],
                           axis=1)


def _edge_blockspecs(widths, be):
    return [pl.BlockSpec((be, w), lambda i: (i, 0)) for w in widths]


def _full_spec(shape):
    nd = len(shape)
    return pl.BlockSpec(shape, lambda i: (0,) * nd)


BE = 2048
BN = 2048


def _tc_call(body, in_arrays, in_specs, out_shapes, out_specs, grid):
    return pl.pallas_call(
        body,
        grid=(grid,),
        in_specs=in_specs,
        out_specs=out_specs,
        out_shape=out_shapes,
    )(*in_arrays)


def _edge1_body(a0, b0, ea, g1, h1, w1ae, b1a, w1b, b1b, w2aeam, w2ae1, b2a,
                e1_o, d2_o, ginv_o):
    xi = a0[:, 0:4]
    gqr = a0[:, 4:8]
    xj = b0[:, 0:4]
    gqc = b0[:, 4:8]
    W = _qmul(ea[...], xi)
    eam = _qmul(_qinv(xj), W)
    ginv_o[...] = _qinv(_qmul(gqc, _qinv(gqr)))
    t = jax.nn.relu(g1[...] + h1[...] + jnp.dot(eam, w1ae[...],
                    preferred_element_type=jnp.float32) + b1a[...])
    e1 = jnp.dot(t, w1b[...], preferred_element_type=jnp.float32) + b1b[...]
    e1_o[...] = e1
    d2_o[...] = (jnp.dot(eam, w2aeam[...], preferred_element_type=jnp.float32)
                 + jnp.dot(jax.nn.relu(e1), w2ae1[...],
                           preferred_element_type=jnp.float32) + b2a[...])


def _edge_mid_body(g, h, d, ep, wkb, bkb, wna_ek, wna_ep, bna, ek_o, dn_o):
    t = jax.nn.relu(g[...] + h[...] + d[...])
    ek = jnp.dot(t, wkb[...], preferred_element_type=jnp.float32) + bkb[...]
    ek_o[...] = ek
    dn_o[...] = (jnp.dot(jax.nn.relu(ek), wna_ek[...],
                         preferred_element_type=jnp.float32)
                 + jnp.dot(jax.nn.relu(ep[...]), wna_ep[...],
                           preferred_element_type=jnp.float32) + bna[...])


def _edge4_body(g, h, d, w4b, b4b, e4_o):
    t = jax.nn.relu(g[...] + h[...] + d[...])
    e4_o[...] = jnp.dot(t, w4b[...], preferred_element_type=jnp.float32) + b4b[...]


def _node1_body(p0, p1, c0, c1, wg, wh, x1_o, g_o, h_o, invc_o):
    cnt = c0[...] + c1[...]
    invc = 1.0 / jnp.maximum(cnt, 1.0)
    invc_o[...] = invc
    x1 = jax.nn.relu((p0[...] + p1[...]) * invc[:, 0:1])
    x1_o[...] = x1
    g_o[...] = jnp.dot(x1, wg[...], preferred_element_type=jnp.float32)
    h_o[...] = jnp.dot(x1, wh[...], preferred_element_type=jnp.float32)


def _node_mid_body(p0, p1, invc, xp, wg_a, wg_b, wh_a, wh_b, xk_o, g_o, h_o):
    xk = jax.nn.relu((p0[...] + p1[...]) * invc[:, 0:1])
    xk_o[...] = xk
    g_o[...] = (jnp.dot(xk, wg_a[...], preferred_element_type=jnp.float32)
                + jnp.dot(xp[...], wg_b[...], preferred_element_type=jnp.float32))
    h_o[...] = (jnp.dot(xk, wh_a[...], preferred_element_type=jnp.float32)
                + jnp.dot(xp[...], wh_b[...], preferred_element_type=jnp.float32))


def _node4_body(p0, p1, invc, xorg, wl, bl, t5_o):
    x4 = jax.nn.relu((p0[...] + p1[...]) * invc[:, 0:1])
    xq = (jnp.dot(x4, wl[...], preferred_element_type=jnp.float32) + bl[...]
          + xorg[...])
    nrm = jnp.sqrt(jnp.sum(xq * xq, axis=1, keepdims=True))
    xn = xq / jnp.maximum(nrm, 1e-12)
    z = jnp.zeros_like(xn[:, 0:3])
    t5_o[...] = jnp.concatenate([xn, invc[:, 0:1], z], axis=1)


def _loss_body(a5, b5, ginv, beta, out):
    i = pl.program_id(0)

    @pl.when(i == 0)
    def _z():
        out[...] = jnp.zeros_like(out)

    x_row = a5[:, 0:4]
    invc_r = a5[:, 4:5]
    x_col = b5[:, 0:4]
    em = _qmul(x_col, _qinv(x_row))
    l1 = _qmul(ginv[...], em)
    nrm = jnp.sqrt(jnp.sum(l1 * l1, axis=1, keepdims=True))
    l1 = l1 / jnp.maximum(nrm, 1e-12)
    alpha = 0.05
    nn0 = jnp.minimum(1.0 - l1[:, 0:1], 1.0 + l1[:, 0:1])
    nnv = (jnp.abs(nn0[:, 0]) + jnp.abs(l1[:, 1]) + jnp.abs(l1[:, 2])
           + jnp.abs(l1[:, 3])) * beta[:, 0]
    le = jnp.where(nnv < alpha, 0.5 * nnv * nnv / alpha, nnv - 0.5 * alpha)
    s = jnp.sum(le * invc_r[:, 0]) * (1.0 / N)
    out[...] += jnp.reshape(s, (1, 1))


_gather4 = _make_gather((8, 8, NF, NF), (0, 1, 0, 1))
_gather2 = _make_gather((NF, NF), (0, 1))
_gather2s = _make_gather((8, 8), (0, 1))
_scatter_c = _make_scatter(True)
_scatter = _make_scatter(False)


def kernel(x_org, edge_index, edge_attr, gt_q, beta,
           W1a, b1a, W1b, b1b, W2a, b2a, W2b, b2b,
           W3a, b3a, W3b, b3b, W4a, b4a, W4b, b4b, Wl, bl):
    f32 = jnp.float32
    row = edge_index[0].astype(jnp.int32)
    col = edge_index[1].astype(jnp.int32)
    padE = EP - E
    rowg = jnp.pad(row, (0, padE)).reshape(NW, NCH, CB)
    colg = jnp.pad(col, (0, padE)).reshape(NW, NCH, CB)
    rows = jnp.pad(row, (0, padE), constant_values=N).reshape(NW, NCH, CB)

    z32 = jnp.zeros((NACC, NF), f32)
    z16 = jnp.zeros((NACC, 16), f32)
    ones16 = jnp.ones((CB, 16), f32)
    eap = jnp.pad(edge_attr[:, :4], ((0, padE), (0, 0)))
    betap = jnp.pad(beta, (0, padE)).reshape(EP, 1)

    T0 = jnp.pad(jnp.concatenate([x_org, gt_q], axis=1),
                 ((0, NACC - N), (0, 0)))
    xorgp = jnp.pad(x_org, ((0, NACC - N), (0, 0)))

    def _prep_body(xo, wg, wh, g_o, h_o):
        g_o[...] = jnp.dot(xo[...], wg[...], preferred_element_type=jnp.float32)
        h_o[...] = jnp.dot(xo[...], wh[...], preferred_element_type=jnp.float32)

    g1t, h1t = _tc_call(
        _prep_body,
        [xorgp, W1a[0:4], W1a[4:8]],
        [pl.BlockSpec((BN, 4), lambda i: (i, 0)), _full_spec((4, NF)),
         _full_spec((4, NF))],
        (jax.ShapeDtypeStruct((NACC, NF), f32),
         jax.ShapeDtypeStruct((NACC, NF), f32)),
        [pl.BlockSpec((BN, NF), lambda i: (i, 0))] * 2,
        NACC // BN)

    A0, B0, G1, H1 = _gather4(T0, T0, g1t, h1t, rowg, colg)

    nb_e = EP // BE
    e1, D2, ginv = _tc_call(
        _edge1_body,
        [A0, B0, eap, G1, H1, W1a[8:12], b1a.reshape(1, NF), W1b,
         b1b.reshape(1, NF), W2a[64:68], W2a[68:100], b2a.reshape(1, NF)],
        _edge_blockspecs([8, 8, 4, NF, NF], BE)
        + [_full_spec((4, NF)), _full_spec((1, NF)), _full_spec((NF, NF)),
           _full_spec((1, NF)), _full_spec((4, NF)), _full_spec((NF, NF)),
           _full_spec((1, NF))],
        (jax.ShapeDtypeStruct((EP, NF), f32), jax.ShapeDtypeStruct((EP, NF), f32),
         jax.ShapeDtypeStruct((EP, 4), f32)),
        _edge_blockspecs([NF, NF, 4], BE),
        nb_e)

    P, C = _scatter_c(e1, rows, z32, z16, ones16)
    x1t, g2t, h2t, invc = _tc_call(
        _node1_body,
        [P[0], P[1], C[0], C[1], W2a[0:32], W2a[32:64]],
        [pl.BlockSpec((BN, NF), lambda i: (i, 0))] * 2
        + [pl.BlockSpec((BN, 16), lambda i: (i, 0))] * 2
        + [_full_spec((NF, NF))] * 2,
        (jax.ShapeDtypeStruct((NACC, NF), f32),
         jax.ShapeDtypeStruct((NACC, NF), f32),
         jax.ShapeDtypeStruct((NACC, NF), f32),
         jax.ShapeDtypeStruct((NACC, 16), f32)),
        [pl.BlockSpec((BN, NF), lambda i: (i, 0))] * 3
        + [pl.BlockSpec((BN, 16), lambda i: (i, 0))],
        NACC // BN)

    eprev = e1
    xprev = x1t
    gt, ht = g2t, h2t
    Ws = {2: (W2b, b2b, W3a[128:160], W3a[160:192], b3a,
              W3a[0:32], W3a[32:64], W3a[64:96], W3a[96:128]),
          3: (W3b, b3b, W4a[128:160], W4a[160:192], b4a,
              W4a[0:32], W4a[32:64], W4a[64:96], W4a[96:128])}
    for k in (2, 3):
        Gk, Hk = _gather2(gt, ht, rowg, colg)
        wkb, bkb, wna_ek, wna_ep, bna, wga, wgb, wha, whb = Ws[k]
        ek, Dn = _tc_call(
            _edge_mid_body,
            [Gk, Hk, D2, eprev, wkb, bkb.reshape(1, NF), wna_ek, wna_ep,
             bna.reshape(1, NF)],
            _edge_blockspecs([NF, NF, NF, NF], BE)
            + [_full_spec((NF, NF)), _full_spec((1, NF)), _full_spec((NF, NF)),
               _full_spec((NF, NF)), _full_spec((1, NF))],
            (jax.ShapeDtypeStruct((EP, NF), f32),
             jax.ShapeDtypeStruct((EP, NF), f32)),
            _edge_blockspecs([NF, NF], BE),
            nb_e)
        Pk = _scatter(ek, rows, z32)[0]
        xk, gnt, hnt = _tc_call(
            _node_mid_body,
            [Pk[0], Pk[1], invc, xprev, wga, wgb, wha, whb],
            [pl.BlockSpec((BN, NF), lambda i: (i, 0))] * 2
            + [pl.BlockSpec((BN, 16), lambda i: (i, 0))]
            + [pl.BlockSpec((BN, NF), lambda i: (i, 0))]
            + [_full_spec((NF, NF))] * 4,
            (jax.ShapeDtypeStruct((NACC, NF), f32),
             jax.ShapeDtypeStruct((NACC, NF), f32),
             jax.ShapeDtypeStruct((NACC, NF), f32)),
            [pl.BlockSpec((BN, NF), lambda i: (i, 0))] * 3,
            NACC // BN)
        eprev = ek
        D2 = Dn
        xprev = xk
        gt, ht = gnt, hnt

    G4, H4 = _gather2(gt, ht, rowg, colg)
    e4 = _tc_call(
        _edge4_body,
        [G4, H4, D2, W4b, b4b.reshape(1, NF)],
        _edge_blockspecs([NF, NF, NF], BE)
        + [_full_spec((NF, NF)), _full_spec((1, NF))],
        jax.ShapeDtypeStruct((EP, NF), f32),
        _edge_blockspecs([NF], BE)[0],
        nb_e)
    P4 = _scatter(e4, rows, z32)[0]
    T5 = _tc_call(
        _node4_body,
        [P4[0], P4[1], invc, xorgp, Wl, bl.reshape(1, 4)],
        [pl.BlockSpec((BN, NF), lambda i: (i, 0))] * 2
        + [pl.BlockSpec((BN, 16), lambda i: (i, 0)),
           pl.BlockSpec((BN, 4), lambda i: (i, 0)),
           _full_spec((NF, 4)), _full_spec((1, 4))],
        jax.ShapeDtypeStruct((NACC, 8), f32),
        pl.BlockSpec((BN, 8), lambda i: (i, 0)),
        NACC // BN)

    A5, B5 = _gather2s(T5, T5, rowg, colg)
    lsum = _tc_call(
        _loss_body,
        [A5, B5, ginv, betap],
        _edge_blockspecs([8, 8, 4, 1], BE),
        jax.ShapeDtypeStruct((1, 1), f32),
        pl.BlockSpec((1, 1), lambda i: (0, 0)),
        nb_e)

    x = T5[:N, 0:4]
    return (x, lsum[0, 0], beta)


# speedup vs baseline: 2.5724x; 1.8817x over previous
import functools
import jax
import jax.numpy as jnp
from jax import lax
from jax.experimental import pallas as pl
from jax.experimental.pallas import tpu as pltpu
from jax.experimental.pallas import tpu_sc as plsc

N = 10000
E = 160000
NF = 32

NC = 2
NS = 16
NW = NC * NS
CB = 128
NCH = 40
EPW = CB * NCH
EP = EPW * NW
NACC = 10240

_mesh = functools.partial(
    plsc.VectorSubcoreMesh,
    core_axis_name="c", subcore_axis_name="s", num_cores=NC, num_subcores=NS)


def _wid():
    return lax.axis_index("s") * NC + lax.axis_index("c")


def _make_gather(dims, sels):
    n = len(dims)
    scratch = [pltpu.VMEM((NCH, CB), jnp.int32), pltpu.VMEM((NCH, CB), jnp.int32)]
    scratch += [pltpu.VMEM((CB, d), jnp.float32) for d in dims]
    scratch += [pltpu.SemaphoreType.DMA for _ in dims]

    def body(*refs):
        tabs = refs[:n]
        idxr_h, idxc_h = refs[n], refs[n + 1]
        outs = refs[n + 2:2 * n + 2]
        idxr_v, idxc_v = refs[2 * n + 2], refs[2 * n + 3]
        bufs = refs[2 * n + 4:3 * n + 4]
        sems = refs[3 * n + 4:4 * n + 4]
        w = _wid()
        pltpu.sync_copy(idxr_h.at[w], idxr_v)
        pltpu.sync_copy(idxc_h.at[w], idxc_v)

        def step(j, carry):
            handles = []
            for i in range(n):
                iv = idxr_v if sels[i] == 0 else idxc_v
                handles.append(
                    pltpu.async_copy(tabs[i].at[iv.at[j]], bufs[i], sems[i]))
            for i in range(n):
                handles[i].wait()
                pltpu.sync_copy(bufs[i], outs[i].at[pl.ds(w * EPW + j * CB, CB)])
            return carry

        lax.fori_loop(0, NCH, step, 0)

    out_type = tuple(jax.ShapeDtypeStruct((EP, d), jnp.float32) for d in dims)
    return pl.kernel(body, out_type=out_type, mesh=_mesh(),
                     scratch_types=tuple(scratch),
                     compiler_params=pltpu.CompilerParams(
                         use_tc_tiling_on_sc=False))


def _make_scatter(with_counts):
    scratch = [
        pltpu.VMEM_SHARED((NACC, NF), jnp.float32),
        pltpu.VMEM((NCH, CB), jnp.int32),
        pltpu.VMEM((CB, NF), jnp.float32),
    ]
    if with_counts:
        scratch += [pltpu.VMEM_SHARED((NACC, 16), jnp.float32),
                    pltpu.VMEM((CB, 16), jnp.float32)]

    def body(*refs):
        if with_counts:
            (vals_h, idx_h, z32_h, z16_h, ones_h, out_h, outc_h,
             acc_sh, idx_v, val_v, accc_sh, ones_v) = refs
        else:
            vals_h, idx_h, z32_h, out_h, acc_sh, idx_v, val_v = refs
        c = lax.axis_index("c")
        s = lax.axis_index("s")
        w = _wid()

        @pl.when(s == 0)
        def _init():
            pltpu.sync_copy(z32_h, acc_sh)
            if with_counts:
                pltpu.sync_copy(z16_h, accc_sh)

        pltpu.sync_copy(idx_h.at[w], idx_v)
        if with_counts:
            pltpu.sync_copy(ones_h, ones_v)
        plsc.subcore_barrier()

        def step(j, carry):
            pltpu.sync_copy(vals_h.at[pl.ds(w * EPW + j * CB, CB)], val_v)
            pltpu.sync_copy(val_v, acc_sh.at[idx_v.at[j]], add=True)
            if with_counts:
                pltpu.sync_copy(ones_v, accc_sh.at[idx_v.at[j]], add=True)
            return carry

        lax.fori_loop(0, NCH, step, 0)
        plsc.subcore_barrier()

        @pl.when(s == 0)
        def _flush():
            pltpu.sync_copy(acc_sh, out_h.at[c])
            if with_counts:
                pltpu.sync_copy(accc_sh, outc_h.at[c])

    out_type = [jax.ShapeDtypeStruct((NC, NACC, NF), jnp.float32)]
    if with_counts:
        out_type.append(jax.ShapeDtypeStruct((NC, NACC, 16), jnp.float32))
    return pl.kernel(body, out_type=tuple(out_type), mesh=_mesh(),
                     scratch_types=tuple(scratch),
                     compiler_params=pltpu.CompilerParams(
                         use_tc_tiling_on_sc=False))


def _col(a, i):
    return a[:, i:i + 1]


def _qmul_t(q, r):
    q0, q1, q2, q3 = q[0:1], q[1:2], q[2:3], q[3:4]
    r0, r1, r2, r3 = r[0:1], r[1:2], r[2:3], r[3:4]
    w = r0 * q0 - r1 * q1 - r2 * q2 - r3 * q3
    x = r0 * q1 + r1 * q0 - r2 * q3 + r3 * q2
    y = r0 * q2 + r1 * q3 + r2 * q0 - r3 * q1
    z = r0 * q3 - r1 * q2 + r2 * q1 + r3 * q0
    return jnp.concatenate([w, x, y, z], axis=0)


def _qinv_t(q):
    return jnp.concatenate([q[0:1], -q[1:2], -q[2:3], -q[3:4]], axis=0)


def _edge_blockspecs(widths, be):
    return [pl.BlockSpec((be, w), lambda i: (i, 0)) for w in widths]


def _full_spec(shape):
    nd = len(shape)
    return pl.BlockSpec(shape, lambda i: (0,) * nd)


BE = 2048
BN = 2048


def _tc_call(body, in_arrays, in_specs, out_shapes, out_specs, grid):
    return pl.pallas_call(
        body,
        grid=(grid,),
        in_specs=in_specs,
        out_specs=out_specs,
        out_shape=out_shapes,
    )(*in_arrays)


def _edge1_body(a0t, b0t, eat, g1, h1, wcomb, b1a, w1b, b1b, w2ae1, b2a,
                e1_o, d2_o, ginv_o):
    xi = a0t[0:4]
    gqr = a0t[4:8]
    xj = b0t[0:4]
    gqc = b0t[4:8]
    W = _qmul_t(eat[...], xi)
    eam = _qmul_t(_qinv_t(xj), W)
    ginv_o[...] = _qinv_t(_qmul_t(gqc, _qinv_t(gqr)))
    eamW = lax.dot_general(eam, wcomb[...], (((0,), (0,)), ((), ())),
                           preferred_element_type=jnp.float32)
    t = jax.nn.relu(g1[...] + h1[...] + eamW[:, 0:NF] + b1a[...])
    e1 = jnp.dot(t, w1b[...], preferred_element_type=jnp.float32) + b1b[...]
    e1_o[...] = e1
    d2_o[...] = (eamW[:, NF:2 * NF]
                 + jnp.dot(jax.nn.relu(e1), w2ae1[...],
                           preferred_element_type=jnp.float32) + b2a[...])


def _edge_mid_body(g, h, d, ep, wkb, bkb, wna_ek, wna_ep, bna, ek_o, dn_o):
    t = jax.nn.relu(g[...] + h[...] + d[...])
    ek = jnp.dot(t, wkb[...], preferred_element_type=jnp.float32) + bkb[...]
    ek_o[...] = ek
    dn_o[...] = (jnp.dot(jax.nn.relu(ek), wna_ek[...],
                         preferred_element_type=jnp.float32)
                 + jnp.dot(jax.nn.relu(ep[...]), wna_ep[...],
                           preferred_element_type=jnp.float32) + bna[...])


def _edge4_body(g, h, d, w4b, b4b, e4_o):
    t = jax.nn.relu(g[...] + h[...] + d[...])
    e4_o[...] = jnp.dot(t, w4b[...], preferred_element_type=jnp.float32) + b4b[...]


def _node1_body(p0, p1, c0, c1, wg, wh, x1_o, g_o, h_o, invc_o):
    cnt = c0[...] + c1[...]
    invc = 1.0 / jnp.maximum(cnt, 1.0)
    invc_o[...] = invc
    x1 = jax.nn.relu((p0[...] + p1[...]) * invc[:, 0:1])
    x1_o[...] = x1
    g_o[...] = jnp.dot(x1, wg[...], preferred_element_type=jnp.float32)
    h_o[...] = jnp.dot(x1, wh[...], preferred_element_type=jnp.float32)


def _node_mid_body(p0, p1, invc, xp, wg_a, wg_b, wh_a, wh_b, xk_o, g_o, h_o):
    xk = jax.nn.relu((p0[...] + p1[...]) * invc[:, 0:1])
    xk_o[...] = xk
    g_o[...] = (jnp.dot(xk, wg_a[...], preferred_element_type=jnp.float32)
                + jnp.dot(xp[...], wg_b[...], preferred_element_type=jnp.float32))
    h_o[...] = (jnp.dot(xk, wh_a[...], preferred_element_type=jnp.float32)
                + jnp.dot(xp[...], wh_b[...], preferred_element_type=jnp.float32))


def _node4_body(p0, p1, invc, xorg, wl, bl, t5_o):
    x4 = jax.nn.relu((p0[...] + p1[...]) * invc[:, 0:1])
    xq = (jnp.dot(x4, wl[...], preferred_element_type=jnp.float32) + bl[...]
          + xorg[...])
    nrm = jnp.sqrt(jnp.sum(xq * xq, axis=1, keepdims=True))
    xn = xq / jnp.maximum(nrm, 1e-12)
    z = jnp.zeros_like(xn[:, 0:3])
    t5_o[...] = jnp.concatenate([xn, invc[:, 0:1], z], axis=1)


def _loss_body(a5t, b5t, ginvt, betat, out):
    i = pl.program_id(0)

    @pl.when(i == 0)
    def _z():
        out[...] = jnp.zeros_like(out)

    x_row = a5t[0:4]
    invc_r = a5t[4:5]
    x_col = b5t[0:4]
    em = _qmul_t(x_col, _qinv_t(x_row))
    l1 = _qmul_t(ginvt[...], em)
    nrm = jnp.sqrt(jnp.sum(l1 * l1, axis=0, keepdims=True))
    l1 = l1 / jnp.maximum(nrm, 1e-12)
    alpha = 0.05
    nn0 = jnp.minimum(1.0 - l1[0:1], 1.0 + l1[0:1])
    nnv = (jnp.abs(nn0) + jnp.abs(l1[1:2]) + jnp.abs(l1[2:3])
           + jnp.abs(l1[3:4])) * betat[...]
    le = jnp.where(nnv < alpha, 0.5 * nnv * nnv / alpha, nnv - 0.5 * alpha)
    s = jnp.sum(le * invc_r) * (1.0 / N)
    out[...] += jnp.reshape(s, (1, 1))


_gather4 = _make_gather((8, 8, NF, NF), (0, 1, 0, 1))
_gather2 = _make_gather((NF, NF), (0, 1))
_gather2s = _make_gather((8, 8), (0, 1))
_scatter_c = _make_scatter(True)
_scatter = _make_scatter(False)


def kernel(x_org, edge_index, edge_attr, gt_q, beta,
           W1a, b1a, W1b, b1b, W2a, b2a, W2b, b2b,
           W3a, b3a, W3b, b3b, W4a, b4a, W4b, b4b, Wl, bl):
    f32 = jnp.float32
    row = edge_index[0].astype(jnp.int32)
    col = edge_index[1].astype(jnp.int32)
    padE = EP - E
    rowg = jnp.pad(row, (0, padE)).reshape(NW, NCH, CB)
    colg = jnp.pad(col, (0, padE)).reshape(NW, NCH, CB)
    rows = jnp.pad(row, (0, padE), constant_values=N).reshape(NW, NCH, CB)

    z32 = jnp.zeros((NACC, NF), f32)
    z16 = jnp.zeros((NACC, 16), f32)
    ones16 = jnp.ones((CB, 16), f32)
    eap = jnp.pad(edge_attr[:, :4], ((0, padE), (0, 0)))
    betap = jnp.pad(beta, (0, padE)).reshape(1, EP)

    T0 = jnp.pad(jnp.concatenate([x_org, gt_q], axis=1),
                 ((0, NACC - N), (0, 0)))
    xorgp = jnp.pad(x_org, ((0, NACC - N), (0, 0)))

    def _prep_body(xo, wg, wh, g_o, h_o):
        g_o[...] = jnp.dot(xo[...], wg[...], preferred_element_type=jnp.float32)
        h_o[...] = jnp.dot(xo[...], wh[...], preferred_element_type=jnp.float32)

    g1t, h1t = _tc_call(
        _prep_body,
        [xorgp, W1a[0:4], W1a[4:8]],
        [pl.BlockSpec((BN, 4), lambda i: (i, 0)), _full_spec((4, NF)),
         _full_spec((4, NF))],
        (jax.ShapeDtypeStruct((NACC, NF), f32),
         jax.ShapeDtypeStruct((NACC, NF), f32)),
        [pl.BlockSpec((BN, NF), lambda i: (i, 0))] * 2,
        NACC // BN)

    A0, B0, G1, H1 = _gather4(T0, T0, g1t, h1t, rowg, colg)

    nb_e = EP // BE
    wcomb = jnp.concatenate([W1a[8:12], W2a[64:68]], axis=1)
    e1, D2, ginvT = _tc_call(
        _edge1_body,
        [A0.T, B0.T, eap.T, G1, H1, wcomb, b1a.reshape(1, NF), W1b,
         b1b.reshape(1, NF), W2a[68:100], b2a.reshape(1, NF)],
        [pl.BlockSpec((8, BE), lambda i: (0, i)),
         pl.BlockSpec((8, BE), lambda i: (0, i)),
         pl.BlockSpec((4, BE), lambda i: (0, i))]
        + _edge_blockspecs([NF, NF], BE)
        + [_full_spec((4, 2 * NF)), _full_spec((1, NF)), _full_spec((NF, NF)),
           _full_spec((1, NF)), _full_spec((NF, NF)), _full_spec((1, NF))],
        (jax.ShapeDtypeStruct((EP, NF), f32), jax.ShapeDtypeStruct((EP, NF), f32),
         jax.ShapeDtypeStruct((4, EP), f32)),
        _edge_blockspecs([NF, NF], BE)
        + [pl.BlockSpec((4, BE), lambda i: (0, i))],
        nb_e)

    P, C = _scatter_c(e1, rows, z32, z16, ones16)
    x1t, g2t, h2t, invc = _tc_call(
        _node1_body,
        [P[0], P[1], C[0], C[1], W2a[0:32], W2a[32:64]],
        [pl.BlockSpec((BN, NF), lambda i: (i, 0))] * 2
        + [pl.BlockSpec((BN, 16), lambda i: (i, 0))] * 2
        + [_full_spec((NF, NF))] * 2,
        (jax.ShapeDtypeStruct((NACC, NF), f32),
         jax.ShapeDtypeStruct((NACC, NF), f32),
         jax.ShapeDtypeStruct((NACC, NF), f32),
         jax.ShapeDtypeStruct((NACC, 16), f32)),
        [pl.BlockSpec((BN, NF), lambda i: (i, 0))] * 3
        + [pl.BlockSpec((BN, 16), lambda i: (i, 0))],
        NACC // BN)

    eprev = e1
    xprev = x1t
    gt, ht = g2t, h2t
    Ws = {2: (W2b, b2b, W3a[128:160], W3a[160:192], b3a,
              W3a[0:32], W3a[32:64], W3a[64:96], W3a[96:128]),
          3: (W3b, b3b, W4a[128:160], W4a[160:192], b4a,
              W4a[0:32], W4a[32:64], W4a[64:96], W4a[96:128])}
    for k in (2, 3):
        Gk, Hk = _gather2(gt, ht, rowg, colg)
        wkb, bkb, wna_ek, wna_ep, bna, wga, wgb, wha, whb = Ws[k]
        ek, Dn = _tc_call(
            _edge_mid_body,
            [Gk, Hk, D2, eprev, wkb, bkb.reshape(1, NF), wna_ek, wna_ep,
             bna.reshape(1, NF)],
            _edge_blockspecs([NF, NF, NF, NF], BE)
            + [_full_spec((NF, NF)), _full_spec((1, NF)), _full_spec((NF, NF)),
               _full_spec((NF, NF)), _full_spec((1, NF))],
            (jax.ShapeDtypeStruct((EP, NF), f32),
             jax.ShapeDtypeStruct((EP, NF), f32)),
            _edge_blockspecs([NF, NF], BE),
            nb_e)
        Pk = _scatter(ek, rows, z32)[0]
        xk, gnt, hnt = _tc_call(
            _node_mid_body,
            [Pk[0], Pk[1], invc, xprev, wga, wgb, wha, whb],
            [pl.BlockSpec((BN, NF), lambda i: (i, 0))] * 2
            + [pl.BlockSpec((BN, 16), lambda i: (i, 0))]
            + [pl.BlockSpec((BN, NF), lambda i: (i, 0))]
            + [_full_spec((NF, NF))] * 4,
            (jax.ShapeDtypeStruct((NACC, NF), f32),
             jax.ShapeDtypeStruct((NACC, NF), f32),
             jax.ShapeDtypeStruct((NACC, NF), f32)),
            [pl.BlockSpec((BN, NF), lambda i: (i, 0))] * 3,
            NACC // BN)
        eprev = ek
        D2 = Dn
        xprev = xk
        gt, ht = gnt, hnt

    G4, H4 = _gather2(gt, ht, rowg, colg)
    e4 = _tc_call(
        _edge4_body,
        [G4, H4, D2, W4b, b4b.reshape(1, NF)],
        _edge_blockspecs([NF, NF, NF], BE)
        + [_full_spec((NF, NF)), _full_spec((1, NF))],
        jax.ShapeDtypeStruct((EP, NF), f32),
        _edge_blockspecs([NF], BE)[0],
        nb_e)
    P4 = _scatter(e4, rows, z32)[0]
    T5 = _tc_call(
        _node4_body,
        [P4[0], P4[1], invc, xorgp, Wl, bl.reshape(1, 4)],
        [pl.BlockSpec((BN, NF), lambda i: (i, 0))] * 2
        + [pl.BlockSpec((BN, 16), lambda i: (i, 0)),
           pl.BlockSpec((BN, 4), lambda i: (i, 0)),
           _full_spec((NF, 4)), _full_spec((1, 4))],
        jax.ShapeDtypeStruct((NACC, 8), f32),
        pl.BlockSpec((BN, 8), lambda i: (i, 0)),
        NACC // BN)

    A5, B5 = _gather2s(T5, T5, rowg, colg)
    lsum = _tc_call(
        _loss_body,
        [A5.T, B5.T, ginvT, betap],
        [pl.BlockSpec((8, BE), lambda i: (0, i)),
         pl.BlockSpec((8, BE), lambda i: (0, i)),
         pl.BlockSpec((4, BE), lambda i: (0, i)),
         pl.BlockSpec((1, BE), lambda i: (0, i))],
        jax.ShapeDtypeStruct((1, 1), f32),
        pl.BlockSpec((1, 1), lambda i: (0, 0)),
        nb_e)

    x = T5[:N, 0:4]
    return (x, lsum[0, 0], beta)
